# baseline (speedup 1.0000x reference)
import functools

import jax
import jax.numpy as jnp
from jax import lax
from jax.experimental import pallas as pl
from jax.experimental.pallas import tpu as pltpu
from jax.experimental.pallas import tpu_sc as plsc

_B, _T, _D, _K = 16, 1024, 64, 8192
_M = _B * _T
_MB = 256
_KH = _K // 2


def _round_f32_to_bf16(v):
    u = lax.bitcast_convert_type(v, jnp.uint32)
    r = (u + jnp.uint32(0x7FFF) + ((u >> jnp.uint32(16)) & jnp.uint32(1)))
    r = r & jnp.uint32(0xFFFF0000)
    return lax.bitcast_convert_type(r, jnp.float32)


def _half_argmin(t_half, offset):
    m = jnp.min(t_half, axis=1, keepdims=True)
    iota = lax.broadcasted_iota(jnp.int32, t_half.shape, 1)
    cand = jnp.where(t_half == m, iota, jnp.int32(_K))
    idx = jnp.min(cand, axis=1, keepdims=True) + jnp.int32(offset)
    return m, idx


def _dist_argmin_body(x_ref, x16_ref, e16_ref, ee_ref, idx_ref):
    xb = x_ref[...]
    p2 = lax.dot_general(x16_ref[...], e16_ref[...], (((1,), (0,)), ((), ())),
                         preferred_element_type=jnp.float32)
    xx = jnp.sum(xb * xb, axis=1, keepdims=True)
    t = (xx - p2) + ee_ref[...]
    m0, idx0 = _half_argmin(t[:, :_KH], 0)
    m1, idx1 = _half_argmin(t[:, _KH:], _KH)
    pick1 = m1 < _round_f32_to_bf16(m0)
    idx_ref[...] = jnp.where(pick1, idx1, idx0)


def _argmin_indices(flat, x16, e16, ee):
    grid = _M // _MB
    out = pl.pallas_call(
        _dist_argmin_body,
        grid=(grid,),
        in_specs=[
            pl.BlockSpec((_MB, _D), lambda i: (i, 0)),
            pl.BlockSpec((_MB, _D), lambda i: (i, 0)),
            pl.BlockSpec((_D, _K), lambda i: (0, 0)),
            pl.BlockSpec((1, _K), lambda i: (0, 0)),
        ],
        out_specs=pl.BlockSpec((_MB, 1), lambda i: (i, 0)),
        out_shape=jax.ShapeDtypeStruct((_M, 1), jnp.int32),
    )(flat, x16, e16, ee)
    return out.reshape(_M)


_DP = 128


@functools.lru_cache(maxsize=1)
def _make_sc_gather():
    info = plsc.get_sparse_core_info()
    nc, ns = info.num_cores, info.num_subcores
    nw = nc * ns
    bpw = _M // nw
    mesh = plsc.VectorSubcoreMesh(core_axis_name="c", subcore_axis_name="s")

    @functools.partial(
        pl.kernel,
        mesh=mesh,
        out_type=jax.ShapeDtypeStruct((_M, _DP), jnp.float32),
        scratch_types=[
            pltpu.VMEM((bpw,), jnp.int32),
            pltpu.VMEM((bpw, _DP), jnp.float32),
            pltpu.SemaphoreType.DMA,
        ],
    )
    def gather_rows(table_hbm, idx_hbm, out_hbm, idx_v, rows_v, sem):
        wid = lax.axis_index("s") * nc + lax.axis_index("c")
        base = wid * bpw
        pltpu.sync_copy(idx_hbm.at[pl.ds(base, bpw)], idx_v)
        pltpu.async_copy(table_hbm.at[idx_v], rows_v, sem).wait()
        pltpu.sync_copy(rows_v, out_hbm.at[pl.ds(base, bpw)])

    return gather_rows


def kernel(x, embed):
    flat = x.reshape(_M, _D)
    embed_t = embed.T
    x16 = flat.astype(jnp.bfloat16)
    e16 = (2.0 * embed_t).astype(jnp.bfloat16)
    ee = jnp.sum(embed_t * embed_t, axis=0, keepdims=True)
    idx = _argmin_indices(flat, x16, e16, ee)
    embed_p = jnp.pad(embed, ((0, 0), (0, _DP - _D)))
    quantize = _make_sc_gather()(embed_p, idx)[:, :_D]
    return quantize.reshape(_B, _T, _D), idx.reshape(_B, _T)


# speedup vs baseline: 1.0948x; 1.0948x over previous
import functools

import jax
import jax.numpy as jnp
from jax import lax
from jax.experimental import pallas as pl
from jax.experimental.pallas import tpu as pltpu
from jax.experimental.pallas import tpu_sc as plsc

_B, _T, _D, _K = 16, 1024, 64, 8192
_M = _B * _T
_MB = 256
_KH = _K // 2


def _round_f32_to_bf16(v):
    u = lax.bitcast_convert_type(v, jnp.uint32)
    r = (u + jnp.uint32(0x7FFF) + ((u >> jnp.uint32(16)) & jnp.uint32(1)))
    r = r & jnp.uint32(0xFFFF0000)
    return lax.bitcast_convert_type(r, jnp.float32)


def _half_argmin(t_half, iota_half, offset):
    m = jnp.min(t_half, axis=1, keepdims=True)
    cand = jnp.where(t_half == m, iota_half, jnp.float32(_K))
    idx = jnp.min(cand, axis=1, keepdims=True).astype(jnp.int32) + jnp.int32(offset)
    return m, idx


def _dist_argmin_body(x_ref, x16_ref, e16_ref, ee_ref, iota_ref, idx_ref):
    xb = x_ref[...]
    p2 = lax.dot_general(x16_ref[...], e16_ref[...], (((1,), (0,)), ((), ())),
                         preferred_element_type=jnp.float32)
    xx = jnp.sum(xb * xb, axis=1, keepdims=True)
    t = (xx - p2) + ee_ref[...]
    iota = iota_ref[...]
    m0, idx0 = _half_argmin(t[:, :_KH], iota, 0)
    m1, idx1 = _half_argmin(t[:, _KH:], iota, _KH)
    pick1 = m1 < _round_f32_to_bf16(m0)
    idx_ref[...] = jnp.where(pick1, idx1, idx0)


def _argmin_indices(flat, x16, e16, ee, iota_f):
    grid = _M // _MB
    out = pl.pallas_call(
        _dist_argmin_body,
        grid=(grid,),
        in_specs=[
            pl.BlockSpec((_MB, _D), lambda i: (i, 0)),
            pl.BlockSpec((_MB, _D), lambda i: (i, 0)),
            pl.BlockSpec((_D, _K), lambda i: (0, 0)),
            pl.BlockSpec((1, _K), lambda i: (0, 0)),
            pl.BlockSpec((1, _KH), lambda i: (0, 0)),
        ],
        out_specs=pl.BlockSpec((_MB, 1), lambda i: (i, 0)),
        out_shape=jax.ShapeDtypeStruct((_M, 1), jnp.int32),
    )(flat, x16, e16, ee, iota_f)
    return out.reshape(_M)


_DP = 128


@functools.lru_cache(maxsize=1)
def _make_sc_gather():
    info = plsc.get_sparse_core_info()
    nc, ns = info.num_cores, info.num_subcores
    nw = nc * ns
    bpw = _M // nw
    mesh = plsc.VectorSubcoreMesh(core_axis_name="c", subcore_axis_name="s")

    @functools.partial(
        pl.kernel,
        mesh=mesh,
        out_type=jax.ShapeDtypeStruct((_M, _DP), jnp.float32),
        scratch_types=[
            pltpu.VMEM((bpw,), jnp.int32),
            pltpu.VMEM((bpw, _DP), jnp.float32),
            pltpu.SemaphoreType.DMA,
        ],
    )
    def gather_rows(table_hbm, idx_hbm, out_hbm, idx_v, rows_v, sem):
        wid = lax.axis_index("s") * nc + lax.axis_index("c")
        base = wid * bpw
        pltpu.sync_copy(idx_hbm.at[pl.ds(base, bpw)], idx_v)
        pltpu.async_copy(table_hbm.at[idx_v], rows_v, sem).wait()
        pltpu.sync_copy(rows_v, out_hbm.at[pl.ds(base, bpw)])

    return gather_rows


def kernel(x, embed):
    flat = x.reshape(_M, _D)
    embed_t = embed.T
    x16 = flat.astype(jnp.bfloat16)
    e16 = (2.0 * embed_t).astype(jnp.bfloat16)
    ee = jnp.sum(embed_t * embed_t, axis=0, keepdims=True)
    iota_f = lax.broadcasted_iota(jnp.float32, (1, _KH), 1)
    idx = _argmin_indices(flat, x16, e16, ee, iota_f)
    embed_p = jnp.pad(embed, ((0, 0), (0, _DP - _D)))
    quantize = _make_sc_gather()(embed_p, idx)[:, :_D]
    return quantize.reshape(_B, _T, _D), idx.reshape(_B, _T)


# speedup vs baseline: 1.1132x; 1.0168x over previous
import functools

import jax
import jax.numpy as jnp
from jax import lax
from jax.experimental import pallas as pl
from jax.experimental.pallas import tpu as pltpu
from jax.experimental.pallas import tpu_sc as plsc

_B, _T, _D, _K = 16, 1024, 64, 8192
_M = _B * _T
_MB = 512
_KH = _K // 2


def _round_f32_to_bf16(v):
    u = lax.bitcast_convert_type(v, jnp.uint32)
    r = (u + jnp.uint32(0x7FFF) + ((u >> jnp.uint32(16)) & jnp.uint32(1)))
    r = r & jnp.uint32(0xFFFF0000)
    return lax.bitcast_convert_type(r, jnp.float32)


def _half_argmin(t_half, iota_half, offset):
    m = jnp.min(t_half, axis=1, keepdims=True)
    cand = jnp.where(t_half == m, iota_half, jnp.float32(_K))
    idx = jnp.min(cand, axis=1, keepdims=True).astype(jnp.int32) + jnp.int32(offset)
    return m, idx


def _dist_argmin_body(x_ref, x16_ref, e16_ref, ee_ref, iota_ref, idx_ref):
    xb = x_ref[...]
    p2 = lax.dot_general(x16_ref[...], e16_ref[...], (((1,), (0,)), ((), ())),
                         preferred_element_type=jnp.float32)
    xx = jnp.sum(xb * xb, axis=1, keepdims=True)
    t = (xx - p2) + ee_ref[...]
    iota = iota_ref[...]
    m0, idx0 = _half_argmin(t[:, :_KH], iota, 0)
    m1, idx1 = _half_argmin(t[:, _KH:], iota, _KH)
    pick1 = m1 < _round_f32_to_bf16(m0)
    idx_ref[...] = jnp.where(pick1, idx1, idx0)


def _argmin_indices(flat, x16, e16, ee, iota_f):
    grid = _M // _MB
    out = pl.pallas_call(
        _dist_argmin_body,
        grid=(grid,),
        in_specs=[
            pl.BlockSpec((_MB, _D), lambda i: (i, 0)),
            pl.BlockSpec((_MB, _D), lambda i: (i, 0)),
            pl.BlockSpec((_D, _K), lambda i: (0, 0)),
            pl.BlockSpec((1, _K), lambda i: (0, 0)),
            pl.BlockSpec((1, _KH), lambda i: (0, 0)),
        ],
        out_specs=pl.BlockSpec((_MB, 1), lambda i: (i, 0)),
        out_shape=jax.ShapeDtypeStruct((_M, 1), jnp.int32),
    )(flat, x16, e16, ee, iota_f)
    return out.reshape(_M)


_DP = 128


@functools.lru_cache(maxsize=1)
def _make_sc_gather():
    info = plsc.get_sparse_core_info()
    nc, ns = info.num_cores, info.num_subcores
    nw = nc * ns
    bpw = _M // nw
    mesh = plsc.VectorSubcoreMesh(core_axis_name="c", subcore_axis_name="s")

    @functools.partial(
        pl.kernel,
        mesh=mesh,
        out_type=jax.ShapeDtypeStruct((_M, _DP), jnp.float32),
        scratch_types=[
            pltpu.VMEM((bpw,), jnp.int32),
            pltpu.VMEM((bpw, _DP), jnp.float32),
            pltpu.SemaphoreType.DMA,
        ],
    )
    def gather_rows(table_hbm, idx_hbm, out_hbm, idx_v, rows_v, sem):
        wid = lax.axis_index("s") * nc + lax.axis_index("c")
        base = wid * bpw
        pltpu.sync_copy(idx_hbm.at[pl.ds(base, bpw)], idx_v)
        pltpu.async_copy(table_hbm.at[idx_v], rows_v, sem).wait()
        pltpu.sync_copy(rows_v, out_hbm.at[pl.ds(base, bpw)])

    return gather_rows


def kernel(x, embed):
    flat = x.reshape(_M, _D)
    embed_t = embed.T
    x16 = flat.astype(jnp.bfloat16)
    e16 = (2.0 * embed_t).astype(jnp.bfloat16)
    ee = jnp.sum(embed_t * embed_t, axis=0, keepdims=True)
    iota_f = lax.broadcasted_iota(jnp.float32, (1, _KH), 1)
    idx = _argmin_indices(flat, x16, e16, ee, iota_f)
    embed_p = jnp.pad(embed, ((0, 0), (0, _DP - _D)))
    quantize = _make_sc_gather()(embed_p, idx)[:, :_D]
    return quantize.reshape(_B, _T, _D), idx.reshape(_B, _T)


# speedup vs baseline: 1.2809x; 1.1506x over previous
import functools

import jax
import jax.numpy as jnp
from jax import lax
from jax.experimental import pallas as pl
from jax.experimental.pallas import tpu as pltpu
from jax.experimental.pallas import tpu_sc as plsc

_B, _T, _D, _K = 16, 1024, 64, 8192
_M = _B * _T
_MB = 512
_KH = _K // 2


def _round_f32_to_bf16(v):
    u = lax.bitcast_convert_type(v, jnp.uint32)
    r = (u + jnp.uint32(0x7FFF) + ((u >> jnp.uint32(16)) & jnp.uint32(1)))
    r = r & jnp.uint32(0xFFFF0000)
    return lax.bitcast_convert_type(r, jnp.float32)


_LW = 128
_NS = _KH // _LW


def _half_argmin(xx, p2, ee, half, lane_iota):
    base = half * _KH
    m_run = (xx - p2[:, base:base + _LW]) + ee[:, base:base + _LW]
    j_run = jnp.zeros_like(m_run)
    for j in range(1, _NS):
        a = base + j * _LW
        tj = (xx - p2[:, a:a + _LW]) + ee[:, a:a + _LW]
        pick = tj < m_run
        m_run = jnp.where(pick, tj, m_run)
        j_run = jnp.where(pick, jnp.float32(j), j_run)
    m = jnp.min(m_run, axis=1, keepdims=True)
    gidx = j_run * jnp.float32(_LW) + lane_iota
    cand = jnp.where(m_run == m, gidx, jnp.float32(_K))
    idx = (jnp.min(cand, axis=1, keepdims=True).astype(jnp.int32)
           + jnp.int32(half * _KH))
    return m, idx


def _dist_argmin_body(x_ref, x16_ref, e16_ref, ee_ref, iota_ref, idx_ref):
    xb = x_ref[...]
    p2 = lax.dot_general(x16_ref[...], e16_ref[...], (((1,), (0,)), ((), ())),
                         preferred_element_type=jnp.float32)
    xx = jnp.sum(xb * xb, axis=1, keepdims=True)
    ee = ee_ref[...]
    lane_iota = iota_ref[...]
    m0, idx0 = _half_argmin(xx, p2, ee, 0, lane_iota)
    m1, idx1 = _half_argmin(xx, p2, ee, 1, lane_iota)
    pick1 = m1 < _round_f32_to_bf16(m0)
    idx_ref[...] = jnp.where(pick1, idx1, idx0)


def _argmin_indices(flat, x16, e16, ee, iota_f):
    grid = _M // _MB
    out = pl.pallas_call(
        _dist_argmin_body,
        grid=(grid,),
        in_specs=[
            pl.BlockSpec((_MB, _D), lambda i: (i, 0)),
            pl.BlockSpec((_MB, _D), lambda i: (i, 0)),
            pl.BlockSpec((_D, _K), lambda i: (0, 0)),
            pl.BlockSpec((1, _K), lambda i: (0, 0)),
            pl.BlockSpec((1, _LW), lambda i: (0, 0)),
        ],
        out_specs=pl.BlockSpec((_MB, 1), lambda i: (i, 0)),
        out_shape=jax.ShapeDtypeStruct((_M, 1), jnp.int32),
    )(flat, x16, e16, ee, iota_f)
    return out.reshape(_M)


_DP = 128


@functools.lru_cache(maxsize=1)
def _make_sc_gather():
    info = plsc.get_sparse_core_info()
    nc, ns = info.num_cores, info.num_subcores
    nw = nc * ns
    bpw = _M // nw
    mesh = plsc.VectorSubcoreMesh(core_axis_name="c", subcore_axis_name="s")

    @functools.partial(
        pl.kernel,
        mesh=mesh,
        out_type=jax.ShapeDtypeStruct((_M, _DP), jnp.float32),
        scratch_types=[
            pltpu.VMEM((bpw,), jnp.int32),
            pltpu.VMEM((bpw, _DP), jnp.float32),
            pltpu.SemaphoreType.DMA,
        ],
    )
    def gather_rows(table_hbm, idx_hbm, out_hbm, idx_v, rows_v, sem):
        wid = lax.axis_index("s") * nc + lax.axis_index("c")
        base = wid * bpw
        pltpu.sync_copy(idx_hbm.at[pl.ds(base, bpw)], idx_v)
        pltpu.async_copy(table_hbm.at[idx_v], rows_v, sem).wait()
        pltpu.sync_copy(rows_v, out_hbm.at[pl.ds(base, bpw)])

    return gather_rows


def kernel(x, embed):
    flat = x.reshape(_M, _D)
    embed_t = embed.T
    x16 = flat.astype(jnp.bfloat16)
    e16 = (2.0 * embed_t).astype(jnp.bfloat16)
    ee = jnp.sum(embed_t * embed_t, axis=0, keepdims=True)
    iota_f = lax.broadcasted_iota(jnp.float32, (1, _LW), 1)
    idx = _argmin_indices(flat, x16, e16, ee, iota_f)
    embed_p = jnp.pad(embed, ((0, 0), (0, _DP - _D)))
    quantize = _make_sc_gather()(embed_p, idx)[:, :_D]
    return quantize.reshape(_B, _T, _D), idx.reshape(_B, _T)
